# baseline (speedup 1.0000x reference)
import functools

import jax
import jax.numpy as jnp
from jax import lax
from jax.experimental import pallas as pl
from jax.experimental.pallas import tpu as pltpu
from jax.experimental.pallas import tpu_sc as plsc

_NC = 2
_NS = 16
_NW = _NC * _NS
_LANES = 16


def _pool_body(b_per_w, S, E, C0, text_hbm, table_hbm, out_hbm,
               idx_v, buf_v, pooled_v, sem):
    wid = lax.axis_index("s") * _NC + lax.axis_index("c")
    base = wid * b_per_w
    C1 = S - C0
    ngrp = E // _LANES

    pltpu.sync_copy(text_hbm.at[pl.ds(base, b_per_w), :], idx_v)

    def row(i, carry):
        h0 = pltpu.async_copy(
            table_hbm.at[idx_v.at[i, pl.ds(0, C0)]],
            buf_v.at[pl.ds(0, C0), :], sem)
        h1 = pltpu.async_copy(
            table_hbm.at[idx_v.at[i, pl.ds(C0, C1)]],
            buf_v.at[pl.ds(C0, C1), :], sem)
        h0.wait()
        h1.wait()

        def acc_body(s, accs):
            return tuple(a + buf_v[s, pl.ds(_LANES * j, _LANES)]
                         for j, a in enumerate(accs))

        accs = lax.fori_loop(
            0, S, acc_body,
            tuple(jnp.zeros((_LANES,), jnp.float32) for _ in range(ngrp)),
            unroll=2)
        for j in range(ngrp):
            pooled_v[i, pl.ds(_LANES * j, _LANES)] = accs[j]
        return carry

    lax.fori_loop(0, b_per_w, row, 0)
    pltpu.sync_copy(pooled_v, out_hbm.at[pl.ds(base, b_per_w), :])


def _mlp_body(x_ref, w1_ref, b1_ref, w2_ref, b2_ref, o_ref):
    h = jnp.tanh(
        jnp.dot(x_ref[...], w1_ref[...], preferred_element_type=jnp.float32)
        + b1_ref[...])
    o_ref[...] = (
        jnp.dot(h, w2_ref[...], preferred_element_type=jnp.float32)
        + b2_ref[...])


def kernel(text, embed_table, W1, b1, W2, b2):
    B, S = text.shape
    V, E = embed_table.shape
    HID = W1.shape[1]
    NCLS = W2.shape[1]
    assert B % _NW == 0 and E % _LANES == 0
    b_per_w = B // _NW
    C0 = min(120, S)

    mesh = plsc.VectorSubcoreMesh(
        core_axis_name="c", subcore_axis_name="s",
        num_cores=_NC, num_subcores=_NS)

    pool = pl.kernel(
        functools.partial(_pool_body, b_per_w, S, E, C0),
        out_type=jax.ShapeDtypeStruct((B, E), jnp.float32),
        mesh=mesh,
        scratch_types=[
            pltpu.VMEM((b_per_w, S), jnp.int32),
            pltpu.VMEM((S, E), jnp.float32),
            pltpu.VMEM((b_per_w, E), jnp.float32),
            pltpu.SemaphoreType.DMA,
        ],
        compiler_params=pltpu.CompilerParams(use_tc_tiling_on_sc=False),
    )
    summed = pool(text.astype(jnp.int32), embed_table)

    w1s = (W1 / S).astype(jnp.float32)
    logits = pl.pallas_call(
        _mlp_body,
        out_shape=jax.ShapeDtypeStruct((B, NCLS), jnp.float32),
    )(summed, w1s, b1.reshape(1, HID), W2, b2.reshape(1, NCLS))
    return logits
